# baseline (speedup 1.0000x reference)
import functools

import jax
import jax.numpy as jnp
from jax.experimental import pallas as pl
from jax.experimental.pallas import tpu as pltpu


def _pick_block(n: int, target: int) -> int:
    best = 0
    for d in range(8, min(n, target) + 1, 8):
        if n % d == 0:
            best = d
    return best if best else n


def _vgae_body(x_ref, w1_ref, wc_ref, adj_ref, muvar_ref, recon_ref,
               h0_ref, h1c_ref, mu_ref, *, p, bm, nhid):
    i = pl.program_id(0)
    row = jax.lax.rem(i, p) * bm

    @pl.when(i == 0)
    def _proj_x():
        h0_ref[...] = jnp.dot(x_ref[...], w1_ref[...],
                              preferred_element_type=jnp.float32)

    @pl.when(i < p)
    def _phase_a():
        acc = jnp.dot(adj_ref[...], h0_ref[...],
                      preferred_element_type=jnp.float32)
        acc = jnp.maximum(acc, 0.0)
        h1c_ref[pl.ds(row, bm), :] = jnp.dot(
            acc, wc_ref[...], preferred_element_type=jnp.float32)

    @pl.when((i >= p) & (i < 2 * p))
    def _phase_b():
        mv = jnp.dot(adj_ref[...], h1c_ref[...],
                     preferred_element_type=jnp.float32)
        muvar_ref[...] = mv
        mu_ref[pl.ds(row, bm), :] = mv[:, :nhid]

    @pl.when(i >= 2 * p)
    def _phase_c():
        recon_ref[...] = jax.lax.dot_general(
            mu_ref[pl.ds(row, bm), :], mu_ref[...],
            (((1,), (1,)), ((), ())),
            preferred_element_type=jnp.float32)


def kernel(x, adj, W1, W2, W3):
    n = adj.shape[0]
    nfeat = x.shape[1]
    nhid = W1.shape[1]
    bm = _pick_block(n, 200)
    p = n // bm

    wc = jnp.concatenate([W2, W3], axis=1)

    def adj_map(i):
        return (jnp.where(i < 2 * p, jax.lax.rem(i, p), p - 1), 0)

    def muvar_map(i):
        return (jnp.clip(i - p, 0, p - 1), 0)

    def recon_map(i):
        return (jnp.clip(i - 2 * p, 0, p - 1), 0)

    muvar, recon = pl.pallas_call(
        functools.partial(_vgae_body, p=p, bm=bm, nhid=nhid),
        grid=(3 * p,),
        in_specs=[
            pl.BlockSpec((n, nfeat), lambda i: (0, 0)),
            pl.BlockSpec((nfeat, nhid), lambda i: (0, 0)),
            pl.BlockSpec((nhid, 2 * nhid), lambda i: (0, 0)),
            pl.BlockSpec((bm, n), adj_map),
        ],
        out_specs=[
            pl.BlockSpec((bm, 2 * nhid), muvar_map),
            pl.BlockSpec((bm, n), recon_map),
        ],
        out_shape=[
            jax.ShapeDtypeStruct((n, 2 * nhid), jnp.float32),
            jax.ShapeDtypeStruct((n, n), jnp.float32),
        ],
        scratch_shapes=[
            pltpu.VMEM((n, nhid), jnp.float32),
            pltpu.VMEM((n, 2 * nhid), jnp.float32),
            pltpu.VMEM((n, nhid), jnp.float32),
        ],
        compiler_params=pltpu.CompilerParams(
            dimension_semantics=("arbitrary",)),
    )(x, W1, wc, adj)

    mu = muvar[:, :nhid]
    logvar = muvar[:, nhid:]
    return (recon, mu, logvar)
